# baseline (speedup 1.0000x reference)
import dataclasses
import functools

import jax
import jax.numpy as jnp
from jax import lax
from jax.experimental import pallas as pl
from jax.experimental.pallas import tpu as pltpu
from jax.experimental.pallas import tpu_sc as plsc

_MAX_STEPS = 65536
_NBINS = _MAX_STEPS + 1
_BINS_PAD = 65664
_N = 1048576
_NC, _NS = 2, 16
_NW = _NC * _NS
_EPT = _N // _NS
_CHUNK = 8192
_NCHUNK = _EPT // _CHUNK


def _sc_compiler_params():
    cp = pltpu.CompilerParams()
    if "needs_layout_passes" in pltpu.CompilerParams.__dataclass_fields__:
        cp = dataclasses.replace(cp, needs_layout_passes=False)
    return cp


def _histograms(halt_steps, rt_true):
    mesh = plsc.VectorSubcoreMesh(core_axis_name="c", subcore_axis_name="s")

    @functools.partial(
        pl.kernel,
        out_type=jax.ShapeDtypeStruct((_NW, _BINS_PAD), jnp.int32),
        mesh=mesh,
        scratch_types=[
            pltpu.VMEM((_BINS_PAD,), jnp.int32),
            pltpu.VMEM((_CHUNK,), jnp.int32),
        ],
        compiler_params=_sc_compiler_params(),
    )
    def hist_kernel(halt_hbm, rt_hbm, out_hbm, hist, buf):
        c = lax.axis_index("c")
        s = lax.axis_index("s")
        wid = c * _NS + s
        base = s * _EPT

        zeros16 = jnp.zeros((16,), jnp.int32)

        @pl.loop(0, _BINS_PAD, step=16)
        def _(i):
            hist[pl.ds(i, 16)] = zeros16

        def process(in_hbm):
            @pl.loop(0, _NCHUNK)
            def _(k):
                pltpu.sync_copy(in_hbm.at[pl.ds(base + k * _CHUNK, _CHUNK)], buf)

                @pl.loop(0, _CHUNK, step=16)
                def _(g):
                    v = buf[pl.ds(g, 16)]
                    cnt, m = plsc.scan_count(v)
                    plsc.addupdate_scatter(hist, [v], cnt, mask=m)

        @pl.when(c == 0)
        def _():
            process(halt_hbm)

        @pl.when(c == 1)
        def _():
            process(rt_hbm)

        pltpu.sync_copy(hist, out_hbm.at[wid])

    return hist_kernel(halt_steps, rt_true)


def _reduce_body(parts_ref, out_ref):
    f = parts_ref[...].astype(jnp.float32)
    p = jnp.sum(f[0:_NS], axis=0)
    t = jnp.sum(f[_NS:_NW], axis=0)
    val = jnp.sum(jnp.exp(t) * (t - p)) * (1.0 / float(_NBINS))
    out_ref[...] = val.reshape(1, 1)


def kernel(halt_steps, rt_true):
    parts = _histograms(halt_steps, rt_true)
    loss = pl.pallas_call(
        _reduce_body,
        out_shape=jax.ShapeDtypeStruct((1, 1), jnp.float32),
    )(parts)
    return loss[0, 0]


# speedup vs baseline: 1.2938x; 1.2938x over previous
import dataclasses
import functools

import jax
import jax.numpy as jnp
from jax import lax
from jax.experimental import pallas as pl
from jax.experimental.pallas import tpu as pltpu
from jax.experimental.pallas import tpu_sc as plsc

_MAX_STEPS = 65536
_NBINS = _MAX_STEPS + 1
_BINS_PAD = 65664
_N = 1048576
_NC, _NS = 2, 16
_NW = _NC * _NS
_EPT = _N // _NS
_CHUNK = 8192
_NCHUNK = _EPT // _CHUNK


def _sc_compiler_params():
    cp = pltpu.CompilerParams()
    if "needs_layout_passes" in pltpu.CompilerParams.__dataclass_fields__:
        cp = dataclasses.replace(cp, needs_layout_passes=False)
    return cp


def _histograms(halt_steps, rt_true):
    mesh = plsc.VectorSubcoreMesh(core_axis_name="c", subcore_axis_name="s")

    @functools.partial(
        pl.kernel,
        out_type=jax.ShapeDtypeStruct((_NW, _BINS_PAD), jnp.int32),
        mesh=mesh,
        scratch_types=[
            pltpu.VMEM((_BINS_PAD,), jnp.int32),
            pltpu.VMEM((_CHUNK,), jnp.int32),
            pltpu.VMEM((_CHUNK,), jnp.int32),
            pltpu.SemaphoreType.DMA,
            pltpu.SemaphoreType.DMA,
        ],
        compiler_params=_sc_compiler_params(),
    )
    def hist_kernel(halt_hbm, rt_hbm, out_hbm, hist, buf0, buf1, sem0, sem1):
        c = lax.axis_index("c")
        s = lax.axis_index("s")
        wid = c * _NS + s
        base = s * _EPT

        zeros16 = jnp.zeros((16,), jnp.int32)

        def scatter_chunk(buf):
            @pl.loop(0, _CHUNK, step=16, unroll=8)
            def _(g):
                v = buf[pl.ds(g, 16)]
                cnt, m = plsc.scan_count(v)
                plsc.addupdate_scatter(hist, [v], cnt, mask=m)

        def process(in_hbm):
            def start(k, buf, sem):
                pltpu.async_copy(in_hbm.at[pl.ds(base + k * _CHUNK, _CHUNK)], buf, sem)

            def wait(buf, sem):
                pltpu.make_async_copy(in_hbm.at[pl.ds(0, _CHUNK)], buf, sem).wait()

            start(0, buf0, sem0)

            @pl.loop(0, _BINS_PAD, step=16, unroll=8)
            def _(i):
                hist[pl.ds(i, 16)] = zeros16

            @pl.loop(0, _NCHUNK, step=2)
            def _(k):
                wait(buf0, sem0)
                start(k + 1, buf1, sem1)
                scatter_chunk(buf0)
                wait(buf1, sem1)

                @pl.when(k + 2 < _NCHUNK)
                def _():
                    start(k + 2, buf0, sem0)

                scatter_chunk(buf1)

        @pl.when(c == 0)
        def _():
            process(halt_hbm)

        @pl.when(c == 1)
        def _():
            process(rt_hbm)

        pltpu.sync_copy(hist, out_hbm.at[wid])

    return hist_kernel(halt_steps, rt_true)


def _reduce_body(parts_ref, out_ref):
    f = parts_ref[...].astype(jnp.float32)
    p = jnp.sum(f[0:_NS], axis=0)
    t = jnp.sum(f[_NS:_NW], axis=0)
    val = jnp.sum(jnp.exp(t) * (t - p)) * (1.0 / float(_NBINS))
    out_ref[...] = val.reshape(1, 1)


def kernel(halt_steps, rt_true):
    parts = _histograms(halt_steps, rt_true)
    loss = pl.pallas_call(
        _reduce_body,
        out_shape=jax.ShapeDtypeStruct((1, 1), jnp.float32),
    )(parts)
    return loss[0, 0]


# speedup vs baseline: 3.1598x; 2.4423x over previous
import dataclasses
import functools

import jax
import jax.numpy as jnp
from jax import lax
from jax.experimental import pallas as pl
from jax.experimental.pallas import tpu as pltpu
from jax.experimental.pallas import tpu_sc as plsc

_MAX_STEPS = 65536
_NBINS = _MAX_STEPS + 1
_BINS_PAD = 65664
_N = 1048576
_NC, _NS = 2, 16
_NW = _NC * _NS
_EPT = _N // _NS
_CHUNK = 8192
_NCHUNK = _EPT // _CHUNK


def _sc_compiler_params():
    cp = pltpu.CompilerParams()
    if "needs_layout_passes" in pltpu.CompilerParams.__dataclass_fields__:
        cp = dataclasses.replace(cp, needs_layout_passes=False)
    return cp


def _histograms(halt_steps, rt_true):
    mesh = plsc.VectorSubcoreMesh(core_axis_name="c", subcore_axis_name="s")

    @functools.partial(
        pl.kernel,
        out_type=jax.ShapeDtypeStruct((_NW, _BINS_PAD), jnp.int32),
        mesh=mesh,
        scratch_types=[
            pltpu.VMEM((_BINS_PAD,), jnp.int32),
            pltpu.VMEM((_CHUNK,), jnp.int32),
            pltpu.VMEM((_CHUNK,), jnp.int32),
            pltpu.SemaphoreType.DMA,
            pltpu.SemaphoreType.DMA,
        ],
        compiler_params=_sc_compiler_params(),
    )
    def hist_kernel(halt_hbm, rt_hbm, out_hbm, hist, buf0, buf1, sem0, sem1):
        c = lax.axis_index("c")
        s = lax.axis_index("s")
        wid = c * _NS + s
        base = s * _EPT

        zeros16 = jnp.zeros((16,), jnp.int32)

        ones16 = jnp.ones((16,), jnp.int32)

        def scatter_chunk(buf):
            @plsc.parallel_loop(0, _CHUNK, step=16, unroll=8)
            def _(g):
                v = buf[pl.ds(g, 16)]
                plsc.addupdate_scatter(hist, [v], ones16)

        def process(in_hbm):
            def start(k, buf, sem):
                pltpu.async_copy(in_hbm.at[pl.ds(base + k * _CHUNK, _CHUNK)], buf, sem)

            def wait(buf, sem):
                pltpu.make_async_copy(in_hbm.at[pl.ds(0, _CHUNK)], buf, sem).wait()

            start(0, buf0, sem0)

            @pl.loop(0, _BINS_PAD, step=16, unroll=8)
            def _(i):
                hist[pl.ds(i, 16)] = zeros16

            @pl.loop(0, _NCHUNK, step=2)
            def _(k):
                wait(buf0, sem0)
                start(k + 1, buf1, sem1)
                scatter_chunk(buf0)
                wait(buf1, sem1)

                @pl.when(k + 2 < _NCHUNK)
                def _():
                    start(k + 2, buf0, sem0)

                scatter_chunk(buf1)

        @pl.when(c == 0)
        def _():
            process(halt_hbm)

        @pl.when(c == 1)
        def _():
            process(rt_hbm)

        pltpu.sync_copy(hist, out_hbm.at[wid])

    return hist_kernel(halt_steps, rt_true)


def _reduce_body(parts_ref, out_ref):
    f = parts_ref[...].astype(jnp.float32)
    p = jnp.sum(f[0:_NS], axis=0)
    t = jnp.sum(f[_NS:_NW], axis=0)
    val = jnp.sum(jnp.exp(t) * (t - p)) * (1.0 / float(_NBINS))
    out_ref[...] = val.reshape(1, 1)


def kernel(halt_steps, rt_true):
    parts = _histograms(halt_steps, rt_true)
    loss = pl.pallas_call(
        _reduce_body,
        out_shape=jax.ShapeDtypeStruct((1, 1), jnp.float32),
    )(parts)
    return loss[0, 0]
